# baseline (speedup 1.0000x reference)
import functools

import jax
import jax.numpy as jnp
from jax import lax
from jax.experimental import pallas as pl
from jax.experimental.pallas import tpu as pltpu
from jax.experimental.pallas import tpu_sc as plsc

N = 10000
D = 128
NC = 2
NS = 16
L = 16
NW = NC * NS

CHUNK = 64
EPW = 10240
NDUMP = 10008

BN = 1000


def _prep_body(x_ref, w1_ref, b1_ref, w2_ref, b2_ref, a1w_ref, a1b_ref,
               a2w_ref, a2b_ref, xj_ref, a1_ref, a2_ref):
    x = x_ref[...]
    dn = (((1,), (1,)), ((), ()))
    xi = lax.dot_general(x, w1_ref[...], dn, preferred_element_type=jnp.float32)
    xi = xi + b1_ref[...]
    xi = jnp.where(xi > 0, xi, 0.2 * xi)
    xj = lax.dot_general(x, w2_ref[...], dn, preferred_element_type=jnp.float32)
    xj = xj + b2_ref[...]
    xj = jnp.where(xj > 0, xj, 0.2 * xj)
    xj_ref[...] = xj
    a1_ref[...] = jnp.sum(xi * a1w_ref[...], axis=1, keepdims=True) + a1b_ref[0, 0]
    a2_ref[...] = jnp.sum(xj * a2w_ref[...], axis=1, keepdims=True) + a2b_ref[0, 0]


def _prep(x0, W1, b1, W2, b2, a1_w, a1_b, a2_w, a2_b):
    full = lambda s: pl.BlockSpec(s, lambda i: (0, 0))
    return pl.pallas_call(
        _prep_body,
        grid=(N // BN,),
        in_specs=[
            pl.BlockSpec((BN, D), lambda i: (i, 0)),
            full((D, D)), full((1, D)), full((D, D)), full((1, D)),
            full((1, D)), full((1, 1)), full((1, D)), full((1, 1)),
        ],
        out_specs=[
            pl.BlockSpec((BN, D), lambda i: (i, 0)),
            pl.BlockSpec((BN, 1), lambda i: (i, 0)),
            pl.BlockSpec((BN, 1), lambda i: (i, 0)),
        ],
        out_shape=[
            jax.ShapeDtypeStruct((N, D), jnp.float32),
            jax.ShapeDtypeStruct((N, 1), jnp.float32),
            jax.ShapeDtypeStruct((N, 1), jnp.float32),
        ],
    )(x0, W1, b1.reshape(1, D), W2, b2.reshape(1, D),
      a1_w, a1_b.reshape(1, 1), a2_w, a2_b.reshape(1, 1))


def _edge_body(xj_hbm, a1_hbm, a2_hbm, pk_hbm, zero_hbm, out_hbm,
               a1_v, a2_v, pk0, pk1, row0, row1, col0, col1, rows0, rows1,
               semg, semp, sems0, sems1, agg_sh):
    pk_b = (pk0, pk1)
    row_b = (row0, row1)
    col_b = (col0, col1)
    rows_b = (rows0, rows1)
    sems = (sems0, sems1)
    c = lax.axis_index("c")
    s = lax.axis_index("s")
    nchunks = EPW // CHUNK
    base = (c * NS + s) * EPW

    pltpu.sync_copy(a1_hbm, a1_v)
    pltpu.sync_copy(a2_hbm, a2_v)

    delta, slab = 624, 640
    r0 = s * delta
    pltpu.sync_copy(zero_hbm.at[pl.ds(r0, slab)], agg_sh.at[pl.ds(r0, slab)])
    plsc.subcore_barrier()

    pltpu.sync_copy(pk_hbm.at[pl.ds(base, CHUNK)], pk0)

    def chunk_step(k, carry):
        off = base + k * CHUNK

        def do(p, o):
            row_v, col_v, rows_v = row_b[p], col_b[p], rows_b[p]

            @pl.when(k >= 1)
            def _():
                pltpu.make_async_copy(pk_hbm.at[pl.ds(0, CHUNK)], pk_b[p],
                                      semp).wait()

            @pl.when(k + 1 < nchunks)
            def _():
                pltpu.async_copy(pk_hbm.at[pl.ds(off + CHUNK, CHUNK)],
                                 pk_b[o], semp)

            for g in range(CHUNK // L):
                sl = pl.ds(g * L, L)
                word = pk_b[p][sl]
                row_v[sl] = word & 0xFFFF
                col_v[sl] = lax.shift_right_logical(word, 16)
            pltpu.async_copy(xj_hbm.at[col_v], rows_v, semg).wait()
            for g in range(CHUNK // L):
                ridx = row_v[pl.ds(g * L, L)]
                cidx = col_v[pl.ds(g * L, L)]
                z = plsc.load_gather(a1_v, [ridx]) + plsc.load_gather(a2_v, [cidx])
                att = 1.0 / (1.0 + jnp.exp(-z))
                for j in range(L):
                    i = g * L + j
                    av = jnp.full((L,), att[j], jnp.float32)
                    for q in range(D // L):
                        sl = pl.ds(q * L, L)
                        rows_v[i, sl] = rows_v[i, sl] * av

            @pl.when(k >= 1)
            def _():
                pltpu.make_async_copy(rows_b[o], agg_sh.at[pl.ds(0, CHUNK)],
                                      sems[o]).wait()

            pltpu.async_copy(rows_v, agg_sh.at[row_v], sems[p], add=True)

        @pl.when(lax.rem(k, 2) == 0)
        def _():
            do(0, 1)

        @pl.when(lax.rem(k, 2) == 1)
        def _():
            do(1, 0)

        return carry

    lax.fori_loop(0, nchunks, chunk_step, 0)
    last = (EPW // CHUNK - 1) % 2
    pltpu.make_async_copy(rows_b[last], agg_sh.at[pl.ds(0, CHUNK)],
                          sems[last]).wait()

    plsc.subcore_barrier()
    pltpu.sync_copy(agg_sh.at[pl.ds(r0, slab)], out_hbm.at[c, pl.ds(r0, slab)])


def _edge_agg(xj, a1, a2, row, col, zero):
    mesh = plsc.VectorSubcoreMesh(core_axis_name="c", subcore_axis_name="s")
    kern = pl.kernel(
        _edge_body,
        out_type=jax.ShapeDtypeStruct((NC, N, D), jnp.float32),
        mesh=mesh,
        compiler_params=pltpu.CompilerParams(needs_layout_passes=False),
        scratch_types=[
            pltpu.VMEM((NDUMP,), jnp.float32),
            pltpu.VMEM((N,), jnp.float32),
            pltpu.VMEM((CHUNK,), jnp.int32),
            pltpu.VMEM((CHUNK,), jnp.int32),
            pltpu.VMEM((CHUNK,), jnp.int32),
            pltpu.VMEM((CHUNK,), jnp.int32),
            pltpu.VMEM((CHUNK,), jnp.int32),
            pltpu.VMEM((CHUNK,), jnp.int32),
            pltpu.VMEM((CHUNK, D), jnp.float32),
            pltpu.VMEM((CHUNK, D), jnp.float32),
            pltpu.SemaphoreType.DMA,
            pltpu.SemaphoreType.DMA,
            pltpu.SemaphoreType.DMA,
            pltpu.SemaphoreType.DMA,
            pltpu.VMEM_SHARED((NDUMP, D), jnp.float32),
        ],
    )
    return kern(xj, a1, a2, row | (col << 16), zero)


def _final_body(p_ref, x_ref, o_ref):
    o_ref[...] = p_ref[0] + p_ref[1] + x_ref[...]


def _finalize(partials, x0):
    return pl.pallas_call(
        _final_body,
        grid=(N // BN,),
        in_specs=[
            pl.BlockSpec((NC, BN, D), lambda i: (0, i, 0)),
            pl.BlockSpec((BN, D), lambda i: (i, 0)),
        ],
        out_specs=pl.BlockSpec((BN, D), lambda i: (i, 0)),
        out_shape=jax.ShapeDtypeStruct((N, D), jnp.float32),
    )(partials, x0)


@jax.jit
def kernel(x0, edge_index, W1, b1, W2, b2, a1_w, a1_b, a2_w, a2_b):
    xj, a1, a2 = _prep(x0, W1, b1, W2, b2, a1_w, a1_b, a2_w, a2_b)
    row = edge_index[0].astype(jnp.int32)
    col = edge_index[1].astype(jnp.int32)
    npad = EPW - row.shape[0] // NW
    roww = jnp.concatenate(
        [row.reshape(NW, -1), jnp.full((NW, npad), N, jnp.int32)], axis=1)
    colw = jnp.concatenate(
        [col.reshape(NW, -1), jnp.zeros((NW, npad), jnp.int32)], axis=1)
    a1p = jnp.concatenate([a1.reshape(N), jnp.zeros((NDUMP - N,), jnp.float32)])
    zero = jnp.zeros((N, D), jnp.float32)
    partials = _edge_agg(xj, a1p, a2.reshape(N), roww.reshape(-1),
                         colw.reshape(-1), zero)
    return _finalize(partials, x0)


# speedup vs baseline: 1.7384x; 1.7384x over previous
import functools

import jax
import jax.numpy as jnp
from jax import lax
from jax.experimental import pallas as pl
from jax.experimental.pallas import tpu as pltpu
from jax.experimental.pallas import tpu_sc as plsc

N = 10000
D = 128
NC = 2
NS = 16
L = 16
NW = NC * NS

CHUNK = 80

BN = 1000


def _prep_body(x_ref, w1_ref, b1_ref, w2_ref, b2_ref, a1w_ref, a1b_ref,
               a2w_ref, a2b_ref, xj_ref, a1_ref, a2_ref):
    x = x_ref[...]
    dn = (((1,), (1,)), ((), ()))
    xi = lax.dot_general(x, w1_ref[...], dn, preferred_element_type=jnp.float32)
    xi = xi + b1_ref[...]
    xi = jnp.where(xi > 0, xi, 0.2 * xi)
    xj = lax.dot_general(x, w2_ref[...], dn, preferred_element_type=jnp.float32)
    xj = xj + b2_ref[...]
    xj = jnp.where(xj > 0, xj, 0.2 * xj)
    xj_ref[...] = xj
    a1_ref[...] = jnp.sum(xi * a1w_ref[...], axis=1, keepdims=True) + a1b_ref[0, 0]
    a2_ref[...] = jnp.sum(xj * a2w_ref[...], axis=1, keepdims=True) + a2b_ref[0, 0]


def _prep(x0, W1, b1, W2, b2, a1_w, a1_b, a2_w, a2_b):
    full = lambda s: pl.BlockSpec(s, lambda i: (0, 0))
    return pl.pallas_call(
        _prep_body,
        grid=(N // BN,),
        in_specs=[
            pl.BlockSpec((BN, D), lambda i: (i, 0)),
            full((D, D)), full((1, D)), full((D, D)), full((1, D)),
            full((1, D)), full((1, 1)), full((1, D)), full((1, 1)),
        ],
        out_specs=[
            pl.BlockSpec((BN, D), lambda i: (i, 0)),
            pl.BlockSpec((BN, 1), lambda i: (i, 0)),
            pl.BlockSpec((BN, 1), lambda i: (i, 0)),
        ],
        out_shape=[
            jax.ShapeDtypeStruct((N, D), jnp.float32),
            jax.ShapeDtypeStruct((N, 1), jnp.float32),
            jax.ShapeDtypeStruct((N, 1), jnp.float32),
        ],
    )(x0, W1, b1.reshape(1, D), W2, b2.reshape(1, D),
      a1_w, a1_b.reshape(1, 1), a2_w, a2_b.reshape(1, 1))


def _edge_body(xj_hbm, a1_hbm, a2_hbm, pk_hbm, zero_hbm, out_hbm,
               a1_v, a2_v, pk0, pk1, row0, row1, col0, col1, rows0, rows1,
               semg, semp, sems0, sems1, agg_sh):
    pk_b = (pk0, pk1)
    row_b = (row0, row1)
    col_b = (col0, col1)
    rows_b = (rows0, rows1)
    sems = (sems0, sems1)
    c = lax.axis_index("c")
    s = lax.axis_index("s")
    n_edges = pk_hbm.shape[0]
    epw = n_edges // NW
    nchunks = epw // CHUNK
    base = (c * NS + s) * epw

    pltpu.sync_copy(a1_hbm, a1_v)
    pltpu.sync_copy(a2_hbm, a2_v)

    delta, slab = 624, 640
    r0 = s * delta
    pltpu.sync_copy(zero_hbm.at[pl.ds(r0, slab)], agg_sh.at[pl.ds(r0, slab)])
    plsc.subcore_barrier()

    pltpu.sync_copy(pk_hbm.at[pl.ds(base, CHUNK)], pk0)

    def chunk_step(k, carry):
        off = base + k * CHUNK

        def do(p, o):
            row_v, col_v, rows_v = row_b[p], col_b[p], rows_b[p]

            @pl.when(k >= 1)
            def _():
                pltpu.make_async_copy(pk_hbm.at[pl.ds(0, CHUNK)], pk_b[p],
                                      semp).wait()

            @pl.when(k + 1 < nchunks)
            def _():
                pltpu.async_copy(pk_hbm.at[pl.ds(off + CHUNK, CHUNK)],
                                 pk_b[o], semp)

            for g in range(CHUNK // L):
                sl = pl.ds(g * L, L)
                word = pk_b[p][sl]
                row_v[sl] = word & 0xFFFF
                col_v[sl] = lax.shift_right_logical(word, 16)
            pltpu.async_copy(xj_hbm.at[col_v], rows_v, semg).wait()
            for g in range(CHUNK // L):
                ridx = row_v[pl.ds(g * L, L)]
                cidx = col_v[pl.ds(g * L, L)]
                z = plsc.load_gather(a1_v, [ridx]) + plsc.load_gather(a2_v, [cidx])
                att = 1.0 / (1.0 + jnp.exp(-z))
                for j in range(L):
                    i = g * L + j
                    av = jnp.full((L,), att[j], jnp.float32)
                    for q in range(D // L):
                        sl = pl.ds(q * L, L)
                        rows_v[i, sl] = rows_v[i, sl] * av

            @pl.when(k >= 1)
            def _():
                pltpu.make_async_copy(rows_b[o], agg_sh.at[pl.ds(0, CHUNK)],
                                      sems[o]).wait()

            pltpu.async_copy(rows_v, agg_sh.at[row_v], sems[p], add=True)

        @pl.when(lax.rem(k, 2) == 0)
        def _():
            do(0, 1)

        @pl.when(lax.rem(k, 2) == 1)
        def _():
            do(1, 0)

        return carry

    lax.fori_loop(0, nchunks, chunk_step, 0)
    pltpu.make_async_copy(rows_b[(125 - 1) % 2], agg_sh.at[pl.ds(0, CHUNK)],
                          sems[(125 - 1) % 2]).wait()

    plsc.subcore_barrier()
    pltpu.sync_copy(agg_sh.at[pl.ds(r0, slab)], out_hbm.at[c, pl.ds(r0, slab)])


def _edge_agg(xj, a1, a2, row, col, zero):
    mesh = plsc.VectorSubcoreMesh(core_axis_name="c", subcore_axis_name="s")
    kern = pl.kernel(
        _edge_body,
        out_type=jax.ShapeDtypeStruct((NC, N, D), jnp.float32),
        mesh=mesh,
        compiler_params=pltpu.CompilerParams(needs_layout_passes=False),
        scratch_types=[
            pltpu.VMEM((N,), jnp.float32),
            pltpu.VMEM((N,), jnp.float32),
            pltpu.VMEM((CHUNK,), jnp.int32),
            pltpu.VMEM((CHUNK,), jnp.int32),
            pltpu.VMEM((CHUNK,), jnp.int32),
            pltpu.VMEM((CHUNK,), jnp.int32),
            pltpu.VMEM((CHUNK,), jnp.int32),
            pltpu.VMEM((CHUNK,), jnp.int32),
            pltpu.VMEM((CHUNK, D), jnp.float32),
            pltpu.VMEM((CHUNK, D), jnp.float32),
            pltpu.SemaphoreType.DMA,
            pltpu.SemaphoreType.DMA,
            pltpu.SemaphoreType.DMA,
            pltpu.SemaphoreType.DMA,
            pltpu.VMEM_SHARED((N, D), jnp.float32),
        ],
    )
    return kern(xj, a1, a2, row | (col << 16), zero)


def _final_body(p_ref, x_ref, o_ref):
    o_ref[...] = p_ref[0] + p_ref[1] + x_ref[...]


def _finalize(partials, x0):
    return pl.pallas_call(
        _final_body,
        grid=(N // BN,),
        in_specs=[
            pl.BlockSpec((NC, BN, D), lambda i: (0, i, 0)),
            pl.BlockSpec((BN, D), lambda i: (i, 0)),
        ],
        out_specs=pl.BlockSpec((BN, D), lambda i: (i, 0)),
        out_shape=jax.ShapeDtypeStruct((N, D), jnp.float32),
    )(partials, x0)


@jax.jit
def kernel(x0, edge_index, W1, b1, W2, b2, a1_w, a1_b, a2_w, a2_b):
    xj, a1, a2 = _prep(x0, W1, b1, W2, b2, a1_w, a1_b, a2_w, a2_b)
    row = edge_index[0].astype(jnp.int32)
    col = edge_index[1].astype(jnp.int32)
    zero = jnp.zeros((N, D), jnp.float32)
    partials = _edge_agg(xj, a1.reshape(N), a2.reshape(N), row, col, zero)
    return _finalize(partials, x0)


# speedup vs baseline: 1.7963x; 1.0333x over previous
import functools

import jax
import jax.numpy as jnp
from jax import lax
from jax.experimental import pallas as pl
from jax.experimental.pallas import tpu as pltpu
from jax.experimental.pallas import tpu_sc as plsc

N = 10000
D = 128
NC = 2
NS = 16
L = 16
NW = NC * NS

CHUNK = 80

BN = 1000


def _prep_body(x_ref, w1_ref, b1_ref, w2_ref, b2_ref, a1w_ref, a1b_ref,
               a2w_ref, a2b_ref, xj_ref, a1_ref, a2_ref):
    x = x_ref[...]
    dn = (((1,), (1,)), ((), ()))
    xi = lax.dot_general(x, w1_ref[...], dn, preferred_element_type=jnp.float32)
    xi = xi + b1_ref[...]
    xi = jnp.where(xi > 0, xi, 0.2 * xi)
    xj = lax.dot_general(x, w2_ref[...], dn, preferred_element_type=jnp.float32)
    xj = xj + b2_ref[...]
    xj = jnp.where(xj > 0, xj, 0.2 * xj)
    xj_ref[...] = xj
    a1_ref[...] = jnp.sum(xi * a1w_ref[...], axis=1, keepdims=True) + a1b_ref[0, 0]
    a2_ref[...] = jnp.sum(xj * a2w_ref[...], axis=1, keepdims=True) + a2b_ref[0, 0]


def _prep(x0, W1, b1, W2, b2, a1_w, a1_b, a2_w, a2_b):
    full = lambda s: pl.BlockSpec(s, lambda i: (0, 0))
    return pl.pallas_call(
        _prep_body,
        grid=(N // BN,),
        in_specs=[
            pl.BlockSpec((BN, D), lambda i: (i, 0)),
            full((D, D)), full((1, D)), full((D, D)), full((1, D)),
            full((1, D)), full((1, 1)), full((1, D)), full((1, 1)),
        ],
        out_specs=[
            pl.BlockSpec((BN, D), lambda i: (i, 0)),
            pl.BlockSpec((BN, 1), lambda i: (i, 0)),
            pl.BlockSpec((BN, 1), lambda i: (i, 0)),
        ],
        out_shape=[
            jax.ShapeDtypeStruct((N, D), jnp.float32),
            jax.ShapeDtypeStruct((N, 1), jnp.float32),
            jax.ShapeDtypeStruct((N, 1), jnp.float32),
        ],
    )(x0, W1, b1.reshape(1, D), W2, b2.reshape(1, D),
      a1_w, a1_b.reshape(1, 1), a2_w, a2_b.reshape(1, 1))


def _edge_body(xj_hbm, a1_hbm, a2_hbm, pk_hbm, zero_hbm, out_hbm,
               a1_v, a2_v, pk0, pk1, row0, row1, col0, col1, rows0, rows1,
               att_v, semg, semp, sems0, sems1, agg_sh):
    pk_b = (pk0, pk1)
    row_b = (row0, row1)
    col_b = (col0, col1)
    rows_b = (rows0, rows1)
    sems = (sems0, sems1)
    c = lax.axis_index("c")
    s = lax.axis_index("s")
    n_edges = pk_hbm.shape[0]
    epw = n_edges // NW
    nchunks = epw // CHUNK
    base = (c * NS + s) * epw

    pltpu.sync_copy(a1_hbm, a1_v)
    pltpu.sync_copy(a2_hbm, a2_v)

    delta, slab = 624, 640
    r0 = s * delta
    pltpu.sync_copy(zero_hbm.at[pl.ds(r0, slab)], agg_sh.at[pl.ds(r0, slab)])
    plsc.subcore_barrier()

    pltpu.sync_copy(pk_hbm.at[pl.ds(base, CHUNK)], pk0)

    def chunk_step(k, carry):
        off = base + k * CHUNK

        def do(p, o):
            row_v, col_v, rows_v = row_b[p], col_b[p], rows_b[p]

            @pl.when(k >= 1)
            def _():
                pltpu.make_async_copy(pk_hbm.at[pl.ds(0, CHUNK)], pk_b[p],
                                      semp).wait()

            @pl.when(k + 1 < nchunks)
            def _():
                pltpu.async_copy(pk_hbm.at[pl.ds(off + CHUNK, CHUNK)],
                                 pk_b[o], semp)

            for g in range(CHUNK // L):
                sl = pl.ds(g * L, L)
                word = pk_b[p][sl]
                row_v[sl] = word & 0xFFFF
                col_v[sl] = lax.shift_right_logical(word, 16)
            pltpu.async_copy(xj_hbm.at[col_v], rows_v, semg)
            for g in range(CHUNK // L):
                sl = pl.ds(g * L, L)
                ridx = row_v[sl]
                cidx = col_v[sl]
                z = plsc.load_gather(a1_v, [ridx]) + plsc.load_gather(a2_v, [cidx])
                att_v[sl] = 1.0 / (1.0 + jnp.exp(-z))
            pltpu.make_async_copy(xj_hbm.at[pl.ds(0, CHUNK)], rows_v,
                                  semg).wait()
            for g in range(CHUNK // L):
                att = att_v[pl.ds(g * L, L)]
                for j in range(L):
                    i = g * L + j
                    av = jnp.full((L,), att[j], jnp.float32)
                    for q in range(D // L):
                        sl = pl.ds(q * L, L)
                        rows_v[i, sl] = rows_v[i, sl] * av

            @pl.when(k >= 1)
            def _():
                pltpu.make_async_copy(rows_b[o], agg_sh.at[pl.ds(0, CHUNK)],
                                      sems[o]).wait()

            pltpu.async_copy(rows_v, agg_sh.at[row_v], sems[p], add=True)

        @pl.when(lax.rem(k, 2) == 0)
        def _():
            do(0, 1)

        @pl.when(lax.rem(k, 2) == 1)
        def _():
            do(1, 0)

        return carry

    lax.fori_loop(0, nchunks, chunk_step, 0)
    pltpu.make_async_copy(rows_b[(125 - 1) % 2], agg_sh.at[pl.ds(0, CHUNK)],
                          sems[(125 - 1) % 2]).wait()

    plsc.subcore_barrier()
    pltpu.sync_copy(agg_sh.at[pl.ds(r0, slab)], out_hbm.at[c, pl.ds(r0, slab)])


def _edge_agg(xj, a1, a2, row, col, zero):
    mesh = plsc.VectorSubcoreMesh(core_axis_name="c", subcore_axis_name="s")
    kern = pl.kernel(
        _edge_body,
        out_type=jax.ShapeDtypeStruct((NC, N, D), jnp.float32),
        mesh=mesh,
        compiler_params=pltpu.CompilerParams(needs_layout_passes=False),
        scratch_types=[
            pltpu.VMEM((N,), jnp.float32),
            pltpu.VMEM((N,), jnp.float32),
            pltpu.VMEM((CHUNK,), jnp.int32),
            pltpu.VMEM((CHUNK,), jnp.int32),
            pltpu.VMEM((CHUNK,), jnp.int32),
            pltpu.VMEM((CHUNK,), jnp.int32),
            pltpu.VMEM((CHUNK,), jnp.int32),
            pltpu.VMEM((CHUNK,), jnp.int32),
            pltpu.VMEM((CHUNK, D), jnp.float32),
            pltpu.VMEM((CHUNK, D), jnp.float32),
            pltpu.VMEM((CHUNK,), jnp.float32),
            pltpu.SemaphoreType.DMA,
            pltpu.SemaphoreType.DMA,
            pltpu.SemaphoreType.DMA,
            pltpu.SemaphoreType.DMA,
            pltpu.VMEM_SHARED((N, D), jnp.float32),
        ],
    )
    return kern(xj, a1, a2, row | (col << 16), zero)


def _final_body(p_ref, x_ref, o_ref):
    o_ref[...] = p_ref[0] + p_ref[1] + x_ref[...]


def _finalize(partials, x0):
    return pl.pallas_call(
        _final_body,
        grid=(N // BN,),
        in_specs=[
            pl.BlockSpec((NC, BN, D), lambda i: (0, i, 0)),
            pl.BlockSpec((BN, D), lambda i: (i, 0)),
        ],
        out_specs=pl.BlockSpec((BN, D), lambda i: (i, 0)),
        out_shape=jax.ShapeDtypeStruct((N, D), jnp.float32),
    )(partials, x0)


@jax.jit
def kernel(x0, edge_index, W1, b1, W2, b2, a1_w, a1_b, a2_w, a2_b):
    xj, a1, a2 = _prep(x0, W1, b1, W2, b2, a1_w, a1_b, a2_w, a2_b)
    row = edge_index[0].astype(jnp.int32)
    col = edge_index[1].astype(jnp.int32)
    zero = jnp.zeros((N, D), jnp.float32)
    partials = _edge_agg(xj, a1.reshape(N), a2.reshape(N), row, col, zero)
    return _finalize(partials, x0)


# speedup vs baseline: 1.8076x; 1.0063x over previous
import functools

import jax
import jax.numpy as jnp
from jax import lax
from jax.experimental import pallas as pl
from jax.experimental.pallas import tpu as pltpu
from jax.experimental.pallas import tpu_sc as plsc

N = 10000
D = 128
NC = 2
NS = 16
L = 16
NW = NC * NS

CHUNK = 80

BN = 1000


def _prep_body(x_ref, w1_ref, b1_ref, w2_ref, b2_ref, a1w_ref, a1b_ref,
               a2w_ref, a2b_ref, xj_ref, a1_ref, a2_ref):
    x = x_ref[...]
    dn = (((1,), (1,)), ((), ()))
    xi = lax.dot_general(x, w1_ref[...], dn, preferred_element_type=jnp.float32)
    xi = xi + b1_ref[...]
    xi = jnp.where(xi > 0, xi, 0.2 * xi)
    xj = lax.dot_general(x, w2_ref[...], dn, preferred_element_type=jnp.float32)
    xj = xj + b2_ref[...]
    xj = jnp.where(xj > 0, xj, 0.2 * xj)
    xj_ref[...] = xj
    a1_ref[...] = jnp.sum(xi * a1w_ref[...], axis=1, keepdims=True) + a1b_ref[0, 0]
    a2_ref[...] = jnp.sum(xj * a2w_ref[...], axis=1, keepdims=True) + a2b_ref[0, 0]


def _prep(x0, W1, b1, W2, b2, a1_w, a1_b, a2_w, a2_b):
    full = lambda s: pl.BlockSpec(s, lambda i: (0, 0))
    return pl.pallas_call(
        _prep_body,
        grid=(N // BN,),
        in_specs=[
            pl.BlockSpec((BN, D), lambda i: (i, 0)),
            full((D, D)), full((1, D)), full((D, D)), full((1, D)),
            full((1, D)), full((1, 1)), full((1, D)), full((1, 1)),
        ],
        out_specs=[
            pl.BlockSpec((BN, D), lambda i: (i, 0)),
            pl.BlockSpec((BN, 1), lambda i: (i, 0)),
            pl.BlockSpec((BN, 1), lambda i: (i, 0)),
        ],
        out_shape=[
            jax.ShapeDtypeStruct((N, D), jnp.float32),
            jax.ShapeDtypeStruct((N, 1), jnp.float32),
            jax.ShapeDtypeStruct((N, 1), jnp.float32),
        ],
    )(x0, W1, b1.reshape(1, D), W2, b2.reshape(1, D),
      a1_w, a1_b.reshape(1, 1), a2_w, a2_b.reshape(1, 1))


def _edge_body(xj_hbm, a1_hbm, a2_hbm, pk_hbm, zero_hbm, out_hbm,
               a1_v, a2_v, pk0, pk1, row0, row1, col0, col1, rows0, rows1,
               att_v, semg, semp, sems0, sems1, agg_sh):
    pk_b = (pk0, pk1)
    row_b = (row0, row1)
    col_b = (col0, col1)
    rows_b = (rows0, rows1)
    sems = (sems0, sems1)
    c = lax.axis_index("c")
    s = lax.axis_index("s")
    n_edges = pk_hbm.shape[0]
    epw = n_edges // NW
    nchunks = epw // CHUNK
    base = (c * NS + s) * epw

    pltpu.sync_copy(a1_hbm, a1_v)
    pltpu.sync_copy(a2_hbm, a2_v)

    delta, slab = 624, 640
    r0 = s * delta
    pltpu.sync_copy(zero_hbm.at[pl.ds(r0, slab)], agg_sh.at[pl.ds(r0, slab)])
    plsc.subcore_barrier()

    pltpu.sync_copy(pk_hbm.at[pl.ds(base, CHUNK)], pk0)
    for g in range(CHUNK // L):
        sl = pl.ds(g * L, L)
        word = pk0[sl]
        row0[sl] = word & 0xFFFF
        col0[sl] = lax.shift_right_logical(word, 16)
    pltpu.async_copy(xj_hbm.at[col0], rows0, semg)
    pltpu.async_copy(pk_hbm.at[pl.ds(base + CHUNK, CHUNK)], pk1, semp)

    def chunk_step(k, carry):
        off = base + k * CHUNK

        def do(p, o):
            row_v, col_v, rows_v = row_b[p], col_b[p], rows_b[p]
            for g in range(CHUNK // L):
                sl = pl.ds(g * L, L)
                ridx = row_v[sl]
                cidx = col_v[sl]
                z = plsc.load_gather(a1_v, [ridx]) + plsc.load_gather(a2_v, [cidx])
                att_v[sl] = 1.0 / (1.0 + jnp.exp(-z))
            pltpu.make_async_copy(xj_hbm.at[pl.ds(0, CHUNK)], rows_v,
                                  semg).wait()
            for g in range(CHUNK // L):
                att = att_v[pl.ds(g * L, L)]
                for j in range(L):
                    i = g * L + j
                    av = jnp.full((L,), att[j], jnp.float32)
                    for q in range(D // L):
                        sl = pl.ds(q * L, L)
                        rows_v[i, sl] = rows_v[i, sl] * av

            @pl.when(k >= 1)
            def _():
                pltpu.make_async_copy(rows_b[o], agg_sh.at[pl.ds(0, CHUNK)],
                                      sems[o]).wait()

            pltpu.async_copy(rows_v, agg_sh.at[row_v], sems[p], add=True)

            @pl.when(k + 1 < nchunks)
            def _():
                pltpu.make_async_copy(pk_hbm.at[pl.ds(0, CHUNK)], pk_b[o],
                                      semp).wait()

                @pl.when(k + 2 < nchunks)
                def _():
                    pltpu.async_copy(pk_hbm.at[pl.ds(off + 2 * CHUNK, CHUNK)],
                                     pk_b[p], semp)

                for g in range(CHUNK // L):
                    sl = pl.ds(g * L, L)
                    word = pk_b[o][sl]
                    row_b[o][sl] = word & 0xFFFF
                    col_b[o][sl] = lax.shift_right_logical(word, 16)
                pltpu.async_copy(xj_hbm.at[col_b[o]], rows_b[o], semg)

        @pl.when(lax.rem(k, 2) == 0)
        def _():
            do(0, 1)

        @pl.when(lax.rem(k, 2) == 1)
        def _():
            do(1, 0)

        return carry

    lax.fori_loop(0, nchunks, chunk_step, 0)
    pltpu.make_async_copy(rows_b[(125 - 1) % 2], agg_sh.at[pl.ds(0, CHUNK)],
                          sems[(125 - 1) % 2]).wait()

    plsc.subcore_barrier()
    pltpu.sync_copy(agg_sh.at[pl.ds(r0, slab)], out_hbm.at[c, pl.ds(r0, slab)])


def _edge_agg(xj, a1, a2, row, col, zero):
    mesh = plsc.VectorSubcoreMesh(core_axis_name="c", subcore_axis_name="s")
    kern = pl.kernel(
        _edge_body,
        out_type=jax.ShapeDtypeStruct((NC, N, D), jnp.float32),
        mesh=mesh,
        compiler_params=pltpu.CompilerParams(needs_layout_passes=False),
        scratch_types=[
            pltpu.VMEM((N,), jnp.float32),
            pltpu.VMEM((N,), jnp.float32),
            pltpu.VMEM((CHUNK,), jnp.int32),
            pltpu.VMEM((CHUNK,), jnp.int32),
            pltpu.VMEM((CHUNK,), jnp.int32),
            pltpu.VMEM((CHUNK,), jnp.int32),
            pltpu.VMEM((CHUNK,), jnp.int32),
            pltpu.VMEM((CHUNK,), jnp.int32),
            pltpu.VMEM((CHUNK, D), jnp.float32),
            pltpu.VMEM((CHUNK, D), jnp.float32),
            pltpu.VMEM((CHUNK,), jnp.float32),
            pltpu.SemaphoreType.DMA,
            pltpu.SemaphoreType.DMA,
            pltpu.SemaphoreType.DMA,
            pltpu.SemaphoreType.DMA,
            pltpu.VMEM_SHARED((N, D), jnp.float32),
        ],
    )
    return kern(xj, a1, a2, row | (col << 16), zero)


def _final_body(p_ref, x_ref, o_ref):
    o_ref[...] = p_ref[0] + p_ref[1] + x_ref[...]


def _finalize(partials, x0):
    return pl.pallas_call(
        _final_body,
        grid=(N // BN,),
        in_specs=[
            pl.BlockSpec((NC, BN, D), lambda i: (0, i, 0)),
            pl.BlockSpec((BN, D), lambda i: (i, 0)),
        ],
        out_specs=pl.BlockSpec((BN, D), lambda i: (i, 0)),
        out_shape=jax.ShapeDtypeStruct((N, D), jnp.float32),
    )(partials, x0)


@jax.jit
def kernel(x0, edge_index, W1, b1, W2, b2, a1_w, a1_b, a2_w, a2_b):
    xj, a1, a2 = _prep(x0, W1, b1, W2, b2, a1_w, a1_b, a2_w, a2_b)
    row = edge_index[0].astype(jnp.int32)
    col = edge_index[1].astype(jnp.int32)
    zero = jnp.zeros((N, D), jnp.float32)
    partials = _edge_agg(xj, a1.reshape(N), a2.reshape(N), row, col, zero)
    return _finalize(partials, x0)
